# baseline (speedup 1.0000x reference)
import functools

import jax
import jax.numpy as jnp
from jax import lax
from jax.experimental import pallas as pl
from jax.experimental.pallas import tpu as pltpu
from jax.experimental.pallas import tpu_sc as plsc

VOCAB = 1000000
EMB = 32
B = 4096
L = 50
TEMP = 0.05

NC, NS = 2, 16
NW = NC * NS
SENTS = 2 * B
S_PER_W = SENTS // NW
IDX_PER_W = S_PER_W * L
IDX_COLS = 128
IDX_ROWS_PER_W = IDX_PER_W // IDX_COLS
CHUNK_S = 64
CHUNK_I = CHUNK_S * L
GROUPS = CHUNK_I // IDX_COLS
N_CHUNKS = S_PER_W // CHUNK_S


LANES = 16
SLAB = 512
N_SLABS = (VOCAB // 128) // (SLAB // 128)
SLABS_PER_W = N_SLABS // NW
TAIL_BASE = (VOCAB // 128) * 128
TAIL_N = VOCAB - TAIL_BASE


def _make_sc_detile():
    mesh = plsc.VectorSubcoreMesh(
        core_axis_name="c", subcore_axis_name="s", num_cores=NC, num_subcores=NS
    )

    @functools.partial(
        pl.kernel,
        out_type=jax.ShapeDtypeStruct((VOCAB * EMB,), jnp.float32),
        mesh=mesh,
        scratch_types=[
            pltpu.VMEM((EMB, SLAB), jnp.float32),
            pltpu.VMEM((EMB, SLAB), jnp.float32),
            pltpu.VMEM((SLAB * EMB,), jnp.float32),
            pltpu.VMEM((SLAB * EMB,), jnp.float32),
            pltpu.SemaphoreType.DMA,
            pltpu.SemaphoreType.DMA,
            pltpu.SemaphoreType.DMA,
            pltpu.SemaphoreType.DMA,
        ],
        compiler_params=pltpu.CompilerParams(
            use_tc_tiling_on_sc=True, needs_layout_passes=False
        ),
    )
    def sc_detile(tt_hbm, tail_hbm, out_hbm, in_a, in_b, out_a, out_b,
                  si_a, si_b, so_a, so_b):
        wid = lax.axis_index("s") * NC + lax.axis_index("c")
        iota = lax.iota(jnp.int32, LANES) * EMB

        def slab_of(k):
            return wid + NW * k

        def fire_in(buf, sem, k):
            pltpu.async_copy(
                tt_hbm.at[:, pl.ds(slab_of(k) * SLAB, SLAB)], buf, sem
            )

        def wait_in(buf, sem):
            pltpu.make_async_copy(tt_hbm.at[:, pl.ds(0, SLAB)], buf, sem).wait()

        def transpose(buf, obuf):
            def per_dim(d, _):
                for g in range(SLAB // LANES):
                    a = buf[d, pl.ds(g * LANES, LANES)]
                    plsc.store_scatter(obuf, [iota + (g * LANES * EMB + d)], a)
                return _
            lax.fori_loop(0, EMB, per_dim, 0)

        def fire_out(obuf, sem, k):
            pltpu.async_copy(
                obuf, out_hbm.at[pl.ds(slab_of(k) * SLAB * EMB, SLAB * EMB)], sem
            )

        def wait_out(obuf, sem):
            pltpu.make_async_copy(
                obuf, out_hbm.at[pl.ds(0, SLAB * EMB)], sem
            ).wait()

        fire_in(in_a, si_a, 0)

        def body(m, _):
            fire_in(in_b, si_b, 2 * m + 1)
            wait_in(in_a, si_a)

            @pl.when(m > 0)
            def _w():
                wait_out(out_a, so_a)

            transpose(in_a, out_a)
            fire_out(out_a, so_a, 2 * m)

            @pl.when(m < (SLABS_PER_W - 1) // 2 - 1)
            def _f():
                fire_in(in_a, si_a, 2 * m + 2)

            wait_in(in_b, si_b)

            @pl.when(m > 0)
            def _w2():
                wait_out(out_b, so_b)

            transpose(in_b, out_b)
            fire_out(out_b, so_b, 2 * m + 1)
            return _

        lax.fori_loop(0, (SLABS_PER_W - 1) // 2, body, 0)

        fire_in(in_a, si_a, SLABS_PER_W - 1)
        wait_in(in_a, si_a)
        wait_out(out_a, so_a)
        transpose(in_a, out_a)
        fire_out(out_a, so_a, SLABS_PER_W - 1)
        wait_out(out_a, so_a)
        wait_out(out_b, so_b)

        @pl.when(wid == 0)
        def _extra():
            pltpu.sync_copy(
                tt_hbm.at[:, pl.ds((N_SLABS - 1) * SLAB, SLAB)], in_a
            )
            transpose(in_a, out_a)
            pltpu.sync_copy(
                out_a, out_hbm.at[pl.ds((N_SLABS - 1) * SLAB * EMB, SLAB * EMB)]
            )

        @pl.when(wid == 1)
        def _tail():
            pltpu.sync_copy(tail_hbm, out_hbm.at[pl.ds(TAIL_BASE * EMB, TAIL_N * EMB)])

    return sc_detile


_sc_detile = _make_sc_detile()


def _make_sc_pool():
    mesh = plsc.VectorSubcoreMesh(
        core_axis_name="c", subcore_axis_name="s", num_cores=NC, num_subcores=NS
    )

    @functools.partial(
        pl.kernel,
        out_type=jax.ShapeDtypeStruct((SENTS, EMB), jnp.float32),
        mesh=mesh,
        scratch_types=[
            pltpu.VMEM((IDX_ROWS_PER_W, IDX_COLS), jnp.int32),
            pltpu.VMEM((CHUNK_I, EMB), jnp.float32),
            pltpu.VMEM((S_PER_W, EMB), jnp.float32),
            pltpu.SemaphoreType.DMA,
        ],
        compiler_params=pltpu.CompilerParams(use_tc_tiling_on_sc=False),
    )
    def sc_pool(sents_hbm, table_hbm, out_hbm, idx_v, rows_v, acc_v, sem):
        wid = lax.axis_index("s") * NC + lax.axis_index("c")
        pltpu.sync_copy(sents_hbm.at[wid], idx_v)
        for k in range(N_CHUNKS):
            def fire(j, _):
                pltpu.async_copy(
                    table_hbm.at[idx_v.at[k * GROUPS + j]],
                    rows_v.at[pl.ds(j * IDX_COLS, IDX_COLS)],
                    sem,
                )
                return _

            lax.fori_loop(0, GROUPS, fire, 0)
            pltpu.make_async_copy(table_hbm.at[pl.ds(0, CHUNK_I)], rows_v, sem).wait()

            def per_sentence(s, _):
                def per_tok(j, carry):
                    a0, a1 = carry
                    r = s * L + j
                    return a0 + rows_v[r, 0:16], a1 + rows_v[r, 16:32]

                a0, a1 = lax.fori_loop(
                    0, L, per_tok,
                    (jnp.zeros(16, jnp.float32), jnp.zeros(16, jnp.float32)),
                )
                acc_v[k * CHUNK_S + s, 0:16] = a0
                acc_v[k * CHUNK_S + s, 16:32] = a1
                return _

            lax.fori_loop(0, CHUNK_S, per_sentence, 0)
        pltpu.sync_copy(acc_v, out_hbm.at[pl.ds(wid * S_PER_W, S_PER_W)])

    return sc_pool


_sc_pool = _make_sc_pool()

BLK = 256


def _tc_loss_body(p1_ref, p2_ref, s1_ref, s2_ref, row0_ref, out_ref):
    i = pl.program_id(0)
    row0 = row0_ref[...]

    cnt2 = jnp.sum((s2_ref[...] == 0).astype(jnp.float32), axis=1, keepdims=True)
    e2 = p2_ref[...] - cnt2 * row0
    n2 = e2 * lax.rsqrt(
        jnp.maximum(jnp.sum(e2 * e2, axis=1, keepdims=True), 1e-24)
    )

    cnt1 = jnp.sum((s1_ref[...] == 0).astype(jnp.float32), axis=1, keepdims=True)
    e1 = p1_ref[...] - cnt1 * row0
    n1 = e1 * lax.rsqrt(
        jnp.maximum(jnp.sum(e1 * e1, axis=1, keepdims=True), 1e-24)
    )

    logits = lax.dot_general(
        n1, n2, (((1,), (1,)), ((), ())),
        precision=lax.Precision.HIGHEST,
        preferred_element_type=jnp.float32,
    ) * (1.0 / TEMP)
    m = jnp.max(logits, axis=1, keepdims=True)
    logz = m[:, 0] + jnp.log(jnp.sum(jnp.exp(logits - m), axis=1))
    col = lax.broadcasted_iota(jnp.int32, logits.shape, 1)
    row = lax.broadcasted_iota(jnp.int32, logits.shape, 0) + i * BLK
    diag = jnp.sum(jnp.where(col == row, logits, 0.0), axis=1)
    part = jnp.sum(logz - diag).reshape(1, 1) * (1.0 / B)

    @pl.when(i == 0)
    def _():
        out_ref[...] = jnp.zeros((1, 1), jnp.float32)

    out_ref[...] += part


_tc_loss = pl.pallas_call(
    _tc_loss_body,
    grid=(B // BLK,),
    in_specs=[
        pl.BlockSpec((BLK, EMB), lambda i: (i, 0)),
        pl.BlockSpec((B, EMB), lambda i: (0, 0)),
        pl.BlockSpec((BLK, L), lambda i: (i, 0)),
        pl.BlockSpec((B, L), lambda i: (0, 0)),
        pl.BlockSpec((1, EMB), lambda i: (0, 0)),
    ],
    out_specs=pl.BlockSpec((1, 1), lambda i: (0, 0)),
    out_shape=jax.ShapeDtypeStruct((1, 1), jnp.float32),
    compiler_params=pltpu.CompilerParams(
        dimension_semantics=("arbitrary",),
    ),
)


def kernel(sentences1, sentences2, emb_table):
    sents = jnp.concatenate([sentences1, sentences2], axis=0).reshape(
        NW, IDX_ROWS_PER_W, IDX_COLS
    )
    tail = lax.slice(emb_table, (TAIL_BASE, 0), (VOCAB, EMB)).reshape(TAIL_N * EMB)
    table_lin = _sc_detile(emb_table.T, tail)
    pooled = _sc_pool(sents, table_lin.reshape(VOCAB, EMB))
    row0 = lax.slice(emb_table, (0, 0), (1, EMB))
    loss = _tc_loss(pooled[:B], pooled[B:], sentences1, sentences2, row0)
    return loss[0, 0]


# speedup vs baseline: 1.0036x; 1.0036x over previous
import functools

import jax
import jax.numpy as jnp
from jax import lax
from jax.experimental import pallas as pl
from jax.experimental.pallas import tpu as pltpu
from jax.experimental.pallas import tpu_sc as plsc

VOCAB = 1000000
EMB = 32
B = 4096
L = 50
TEMP = 0.05

NC, NS = 2, 16
NW = NC * NS
SENTS = 2 * B
S_PER_W = SENTS // NW
IDX_PER_W = S_PER_W * L
IDX_COLS = 128
IDX_ROWS_PER_W = IDX_PER_W // IDX_COLS
CHUNK_S = 64
CHUNK_I = CHUNK_S * L
GROUPS = CHUNK_I // IDX_COLS
N_CHUNKS = S_PER_W // CHUNK_S


LANES = 16
SLAB = 512
N_SLABS = (VOCAB // 128) // (SLAB // 128)
SLABS_PER_W = N_SLABS // NW
TAIL_BASE = (VOCAB // 128) * 128
TAIL_N = VOCAB - TAIL_BASE


def _make_sc_detile():
    mesh = plsc.VectorSubcoreMesh(
        core_axis_name="c", subcore_axis_name="s", num_cores=NC, num_subcores=NS
    )

    @functools.partial(
        pl.kernel,
        out_type=jax.ShapeDtypeStruct((VOCAB * EMB,), jnp.float32),
        mesh=mesh,
        scratch_types=[
            pltpu.VMEM((EMB, SLAB), jnp.float32),
            pltpu.VMEM((EMB, SLAB), jnp.float32),
            pltpu.VMEM((SLAB * EMB,), jnp.float32),
            pltpu.VMEM((SLAB * EMB,), jnp.float32),
            pltpu.SemaphoreType.DMA,
            pltpu.SemaphoreType.DMA,
            pltpu.SemaphoreType.DMA,
            pltpu.SemaphoreType.DMA,
        ],
        compiler_params=pltpu.CompilerParams(
            use_tc_tiling_on_sc=True, needs_layout_passes=False
        ),
    )
    def sc_detile(tt_hbm, tail_hbm, out_hbm, in_a, in_b, out_a, out_b,
                  si_a, si_b, so_a, so_b):
        wid = lax.axis_index("s") * NC + lax.axis_index("c")
        iota = lax.iota(jnp.int32, LANES) * EMB

        def slab_of(k):
            return wid + NW * k

        def fire_in(buf, sem, k):
            pltpu.async_copy(
                tt_hbm.at[:, pl.ds(slab_of(k) * SLAB, SLAB)], buf, sem
            )

        def wait_in(buf, sem):
            pltpu.make_async_copy(tt_hbm.at[:, pl.ds(0, SLAB)], buf, sem).wait()

        def transpose(buf, obuf):
            def per_dim(d, _):
                for g0 in range(0, SLAB // LANES, 8):
                    vals = [buf[d, pl.ds((g0 + j) * LANES, LANES)] for j in range(8)]
                    idxs = [iota + ((g0 + j) * LANES * EMB + d) for j in range(8)]
                    for j in range(8):
                        plsc.store_scatter(obuf, [idxs[j]], vals[j])
                return _
            lax.fori_loop(0, EMB, per_dim, 0)

        def fire_out(obuf, sem, k):
            pltpu.async_copy(
                obuf, out_hbm.at[pl.ds(slab_of(k) * SLAB * EMB, SLAB * EMB)], sem
            )

        def wait_out(obuf, sem):
            pltpu.make_async_copy(
                obuf, out_hbm.at[pl.ds(0, SLAB * EMB)], sem
            ).wait()

        fire_in(in_a, si_a, 0)

        def body(m, _):
            fire_in(in_b, si_b, 2 * m + 1)
            wait_in(in_a, si_a)

            @pl.when(m > 0)
            def _w():
                wait_out(out_a, so_a)

            transpose(in_a, out_a)
            fire_out(out_a, so_a, 2 * m)

            @pl.when(m < (SLABS_PER_W - 1) // 2 - 1)
            def _f():
                fire_in(in_a, si_a, 2 * m + 2)

            wait_in(in_b, si_b)

            @pl.when(m > 0)
            def _w2():
                wait_out(out_b, so_b)

            transpose(in_b, out_b)
            fire_out(out_b, so_b, 2 * m + 1)
            return _

        lax.fori_loop(0, (SLABS_PER_W - 1) // 2, body, 0)

        fire_in(in_a, si_a, SLABS_PER_W - 1)
        wait_in(in_a, si_a)
        wait_out(out_a, so_a)
        transpose(in_a, out_a)
        fire_out(out_a, so_a, SLABS_PER_W - 1)
        wait_out(out_a, so_a)
        wait_out(out_b, so_b)

        @pl.when(wid == 0)
        def _extra():
            pltpu.sync_copy(
                tt_hbm.at[:, pl.ds((N_SLABS - 1) * SLAB, SLAB)], in_a
            )
            transpose(in_a, out_a)
            pltpu.sync_copy(
                out_a, out_hbm.at[pl.ds((N_SLABS - 1) * SLAB * EMB, SLAB * EMB)]
            )

        @pl.when(wid == 1)
        def _tail():
            pltpu.sync_copy(tail_hbm, out_hbm.at[pl.ds(TAIL_BASE * EMB, TAIL_N * EMB)])

    return sc_detile


_sc_detile = _make_sc_detile()


def _make_sc_pool():
    mesh = plsc.VectorSubcoreMesh(
        core_axis_name="c", subcore_axis_name="s", num_cores=NC, num_subcores=NS
    )

    @functools.partial(
        pl.kernel,
        out_type=jax.ShapeDtypeStruct((SENTS, EMB), jnp.float32),
        mesh=mesh,
        scratch_types=[
            pltpu.VMEM((IDX_ROWS_PER_W, IDX_COLS), jnp.int32),
            pltpu.VMEM((CHUNK_I, EMB), jnp.float32),
            pltpu.VMEM((S_PER_W, EMB), jnp.float32),
            pltpu.SemaphoreType.DMA,
        ],
        compiler_params=pltpu.CompilerParams(use_tc_tiling_on_sc=False),
    )
    def sc_pool(sents_hbm, table_hbm, out_hbm, idx_v, rows_v, acc_v, sem):
        wid = lax.axis_index("s") * NC + lax.axis_index("c")
        pltpu.sync_copy(sents_hbm.at[wid], idx_v)
        for k in range(N_CHUNKS):
            def fire(j, _):
                pltpu.async_copy(
                    table_hbm.at[idx_v.at[k * GROUPS + j]],
                    rows_v.at[pl.ds(j * IDX_COLS, IDX_COLS)],
                    sem,
                )
                return _

            lax.fori_loop(0, GROUPS, fire, 0)
            pltpu.make_async_copy(table_hbm.at[pl.ds(0, CHUNK_I)], rows_v, sem).wait()

            def per_sentence(s, _):
                def per_tok(j, carry):
                    a0, a1 = carry
                    r = s * L + j
                    return a0 + rows_v[r, 0:16], a1 + rows_v[r, 16:32]

                a0, a1 = lax.fori_loop(
                    0, L, per_tok,
                    (jnp.zeros(16, jnp.float32), jnp.zeros(16, jnp.float32)),
                )
                acc_v[k * CHUNK_S + s, 0:16] = a0
                acc_v[k * CHUNK_S + s, 16:32] = a1
                return _

            lax.fori_loop(0, CHUNK_S, per_sentence, 0)
        pltpu.sync_copy(acc_v, out_hbm.at[pl.ds(wid * S_PER_W, S_PER_W)])

    return sc_pool


_sc_pool = _make_sc_pool()

BLK = 256


def _tc_loss_body(p1_ref, p2_ref, s1_ref, s2_ref, row0_ref, out_ref):
    i = pl.program_id(0)
    row0 = row0_ref[...]

    cnt2 = jnp.sum((s2_ref[...] == 0).astype(jnp.float32), axis=1, keepdims=True)
    e2 = p2_ref[...] - cnt2 * row0
    n2 = e2 * lax.rsqrt(
        jnp.maximum(jnp.sum(e2 * e2, axis=1, keepdims=True), 1e-24)
    )

    cnt1 = jnp.sum((s1_ref[...] == 0).astype(jnp.float32), axis=1, keepdims=True)
    e1 = p1_ref[...] - cnt1 * row0
    n1 = e1 * lax.rsqrt(
        jnp.maximum(jnp.sum(e1 * e1, axis=1, keepdims=True), 1e-24)
    )

    logits = lax.dot_general(
        n1, n2, (((1,), (1,)), ((), ())),
        precision=lax.Precision.HIGHEST,
        preferred_element_type=jnp.float32,
    ) * (1.0 / TEMP)
    m = jnp.max(logits, axis=1, keepdims=True)
    logz = m[:, 0] + jnp.log(jnp.sum(jnp.exp(logits - m), axis=1))
    col = lax.broadcasted_iota(jnp.int32, logits.shape, 1)
    row = lax.broadcasted_iota(jnp.int32, logits.shape, 0) + i * BLK
    diag = jnp.sum(jnp.where(col == row, logits, 0.0), axis=1)
    part = jnp.sum(logz - diag).reshape(1, 1) * (1.0 / B)

    @pl.when(i == 0)
    def _():
        out_ref[...] = jnp.zeros((1, 1), jnp.float32)

    out_ref[...] += part


_tc_loss = pl.pallas_call(
    _tc_loss_body,
    grid=(B // BLK,),
    in_specs=[
        pl.BlockSpec((BLK, EMB), lambda i: (i, 0)),
        pl.BlockSpec((B, EMB), lambda i: (0, 0)),
        pl.BlockSpec((BLK, L), lambda i: (i, 0)),
        pl.BlockSpec((B, L), lambda i: (0, 0)),
        pl.BlockSpec((1, EMB), lambda i: (0, 0)),
    ],
    out_specs=pl.BlockSpec((1, 1), lambda i: (0, 0)),
    out_shape=jax.ShapeDtypeStruct((1, 1), jnp.float32),
    compiler_params=pltpu.CompilerParams(
        dimension_semantics=("arbitrary",),
    ),
)


def kernel(sentences1, sentences2, emb_table):
    sents = jnp.concatenate([sentences1, sentences2], axis=0).reshape(
        NW, IDX_ROWS_PER_W, IDX_COLS
    )
    tail = lax.slice(emb_table, (TAIL_BASE, 0), (VOCAB, EMB)).reshape(TAIL_N * EMB)
    table_lin = _sc_detile(emb_table.T, tail)
    pooled = _sc_pool(sents, table_lin.reshape(VOCAB, EMB))
    row0 = lax.slice(emb_table, (0, 0), (1, EMB))
    loss = _tc_loss(pooled[:B], pooled[B:], sentences1, sentences2, row0)
    return loss[0, 0]


# speedup vs baseline: 2.0831x; 2.0756x over previous
import functools

import jax
import jax.numpy as jnp
from jax import lax
from jax.experimental import pallas as pl
from jax.experimental.pallas import tpu as pltpu
from jax.experimental.pallas import tpu_sc as plsc

VOCAB = 1000000
EMB = 32
EMBP = 40
B = 4096
L = 50
TEMP = 0.05

NC, NS = 2, 16
NW = NC * NS
SENTS = 2 * B
S_PER_W = SENTS // NW
IDX_PER_W = S_PER_W * L
IDX_COLS = 64
IDX_ROWS_PER_W = IDX_PER_W // IDX_COLS
CHUNK_S = 32
CHUNK_I = CHUNK_S * L
GROUPS = CHUNK_I // IDX_COLS
N_CHUNKS = S_PER_W // CHUNK_S

LANES = 16
SLAB = 512
N_SLABS = (VOCAB // 128) // (SLAB // 128)
SLABS_PER_W = N_SLABS // NW
TAIL_BASE = (VOCAB // 128) * 128
TAIL_N = VOCAB - TAIL_BASE


def _make_sc_detile():
    mesh = plsc.VectorSubcoreMesh(
        core_axis_name="c", subcore_axis_name="s", num_cores=NC, num_subcores=NS
    )

    @functools.partial(
        pl.kernel,
        out_type=jax.ShapeDtypeStruct((VOCAB * EMBP,), jnp.float32),
        mesh=mesh,
        scratch_types=[
            pltpu.VMEM((EMB, SLAB), jnp.float32),
            pltpu.VMEM((EMB, SLAB), jnp.float32),
            pltpu.VMEM((SLAB * EMBP,), jnp.float32),
            pltpu.VMEM((SLAB * EMBP,), jnp.float32),
            pltpu.SemaphoreType.DMA,
            pltpu.SemaphoreType.DMA,
            pltpu.SemaphoreType.DMA,
            pltpu.SemaphoreType.DMA,
        ],
        compiler_params=pltpu.CompilerParams(
            use_tc_tiling_on_sc=True, needs_layout_passes=False
        ),
    )
    def sc_detile(tt_hbm, tail_hbm, out_hbm, in_a, in_b, out_a, out_b,
                  si_a, si_b, so_a, so_b):
        wid = lax.axis_index("s") * NC + lax.axis_index("c")
        iota = lax.iota(jnp.int32, LANES) * EMBP

        def slab_of(k):
            return wid + NW * k

        def fire_in(buf, sem, k):
            pltpu.async_copy(
                tt_hbm.at[:, pl.ds(slab_of(k) * SLAB, SLAB)], buf, sem
            )

        def wait_in(buf, sem):
            pltpu.make_async_copy(tt_hbm.at[:, pl.ds(0, SLAB)], buf, sem).wait()

        def transpose(buf, obuf):
            def per_dim(d, _):
                for g0 in range(0, SLAB // LANES, 8):
                    vals = [buf[d, pl.ds((g0 + j) * LANES, LANES)] for j in range(8)]
                    idxs = [iota + ((g0 + j) * LANES * EMBP + d) for j in range(8)]
                    for j in range(8):
                        plsc.store_scatter(obuf, [idxs[j]], vals[j])
                return _
            lax.fori_loop(0, EMB, per_dim, 0)

        def fire_out(obuf, sem, k):
            pltpu.async_copy(
                obuf, out_hbm.at[pl.ds(slab_of(k) * SLAB * EMBP, SLAB * EMBP)], sem
            )

        def wait_out(obuf, sem):
            pltpu.make_async_copy(
                obuf, out_hbm.at[pl.ds(0, SLAB * EMBP)], sem
            ).wait()

        fire_in(in_a, si_a, 0)

        def body(m, _):
            fire_in(in_b, si_b, 2 * m + 1)
            wait_in(in_a, si_a)

            @pl.when(m > 0)
            def _w():
                wait_out(out_a, so_a)

            transpose(in_a, out_a)
            fire_out(out_a, so_a, 2 * m)

            @pl.when(m < (SLABS_PER_W - 1) // 2 - 1)
            def _f():
                fire_in(in_a, si_a, 2 * m + 2)

            wait_in(in_b, si_b)

            @pl.when(m > 0)
            def _w2():
                wait_out(out_b, so_b)

            transpose(in_b, out_b)
            fire_out(out_b, so_b, 2 * m + 1)
            return _

        lax.fori_loop(0, (SLABS_PER_W - 1) // 2, body, 0)

        fire_in(in_a, si_a, SLABS_PER_W - 1)
        wait_in(in_a, si_a)
        wait_out(out_a, so_a)
        transpose(in_a, out_a)
        fire_out(out_a, so_a, SLABS_PER_W - 1)
        wait_out(out_a, so_a)
        wait_out(out_b, so_b)

        @pl.when(wid == 0)
        def _extra():
            pltpu.sync_copy(
                tt_hbm.at[:, pl.ds((N_SLABS - 1) * SLAB, SLAB)], in_a
            )
            transpose(in_a, out_a)
            pltpu.sync_copy(
                out_a, out_hbm.at[pl.ds((N_SLABS - 1) * SLAB * EMBP, SLAB * EMBP)]
            )

        @pl.when(wid == 1)
        def _tail():
            pltpu.sync_copy(tail_hbm, out_b.at[pl.ds(0, TAIL_N * EMBP)])
            pltpu.sync_copy(
                out_b.at[pl.ds(0, TAIL_N * EMBP)],
                out_hbm.at[pl.ds(TAIL_BASE * EMBP, TAIL_N * EMBP)],
            )

    return sc_detile


_sc_detile = _make_sc_detile()


def _make_sc_pool():
    mesh = plsc.VectorSubcoreMesh(
        core_axis_name="c", subcore_axis_name="s", num_cores=NC, num_subcores=NS
    )

    @functools.partial(
        pl.kernel,
        out_type=jax.ShapeDtypeStruct((SENTS, EMB), jnp.float32),
        mesh=mesh,
        scratch_types=[
            pltpu.VMEM((IDX_ROWS_PER_W, IDX_COLS), jnp.int32),
            pltpu.VMEM((CHUNK_I, EMBP), jnp.float32),
            pltpu.VMEM((S_PER_W, EMB), jnp.float32),
            pltpu.SemaphoreType.DMA,
        ],
        compiler_params=pltpu.CompilerParams(use_tc_tiling_on_sc=False),
    )
    def sc_pool(sents_hbm, table_hbm, out_hbm, idx_v, rows_v, acc_v, sem):
        wid = lax.axis_index("s") * NC + lax.axis_index("c")
        pltpu.sync_copy(sents_hbm.at[wid], idx_v)
        for k in range(N_CHUNKS):
            def fire(j, _):
                pltpu.async_copy(
                    table_hbm.at[idx_v.at[k * GROUPS + j]],
                    rows_v.at[pl.ds(j * IDX_COLS, IDX_COLS)],
                    sem,
                )
                return _

            lax.fori_loop(0, GROUPS, fire, 0)
            pltpu.make_async_copy(table_hbm.at[pl.ds(0, CHUNK_I)], rows_v, sem).wait()

            def per_sentence(s, _):
                def per_tok(j, carry):
                    a0, a1 = carry
                    r = s * L + j
                    return a0 + rows_v[r, 0:16], a1 + rows_v[r, 16:32]

                a0, a1 = lax.fori_loop(
                    0, L, per_tok,
                    (jnp.zeros(16, jnp.float32), jnp.zeros(16, jnp.float32)),
                )
                acc_v[k * CHUNK_S + s, 0:16] = a0
                acc_v[k * CHUNK_S + s, 16:32] = a1
                return _

            lax.fori_loop(0, CHUNK_S, per_sentence, 0)
        pltpu.sync_copy(acc_v, out_hbm.at[pl.ds(wid * S_PER_W, S_PER_W)])

    return sc_pool


_sc_pool = _make_sc_pool()

BLK = 256


def _tc_loss_body(p1_ref, p2_ref, s1_ref, s2_ref, row0_ref, out_ref):
    i = pl.program_id(0)
    row0 = row0_ref[...]

    cnt2 = jnp.sum((s2_ref[...] == 0).astype(jnp.float32), axis=1, keepdims=True)
    e2 = p2_ref[...] - cnt2 * row0
    n2 = e2 * lax.rsqrt(
        jnp.maximum(jnp.sum(e2 * e2, axis=1, keepdims=True), 1e-24)
    )

    cnt1 = jnp.sum((s1_ref[...] == 0).astype(jnp.float32), axis=1, keepdims=True)
    e1 = p1_ref[...] - cnt1 * row0
    n1 = e1 * lax.rsqrt(
        jnp.maximum(jnp.sum(e1 * e1, axis=1, keepdims=True), 1e-24)
    )

    logits = lax.dot_general(
        n1, n2, (((1,), (1,)), ((), ())),
        precision=lax.Precision.HIGHEST,
        preferred_element_type=jnp.float32,
    ) * (1.0 / TEMP)
    m = jnp.max(logits, axis=1, keepdims=True)
    logz = m[:, 0] + jnp.log(jnp.sum(jnp.exp(logits - m), axis=1))
    col = lax.broadcasted_iota(jnp.int32, logits.shape, 1)
    row = lax.broadcasted_iota(jnp.int32, logits.shape, 0) + i * BLK
    diag = jnp.sum(jnp.where(col == row, logits, 0.0), axis=1)
    part = jnp.sum(logz - diag).reshape(1, 1) * (1.0 / B)

    @pl.when(i == 0)
    def _():
        out_ref[...] = jnp.zeros((1, 1), jnp.float32)

    out_ref[...] += part


_tc_loss = pl.pallas_call(
    _tc_loss_body,
    grid=(B // BLK,),
    in_specs=[
        pl.BlockSpec((BLK, EMB), lambda i: (i, 0)),
        pl.BlockSpec((B, EMB), lambda i: (0, 0)),
        pl.BlockSpec((BLK, L), lambda i: (i, 0)),
        pl.BlockSpec((B, L), lambda i: (0, 0)),
        pl.BlockSpec((1, EMB), lambda i: (0, 0)),
    ],
    out_specs=pl.BlockSpec((1, 1), lambda i: (0, 0)),
    out_shape=jax.ShapeDtypeStruct((1, 1), jnp.float32),
    compiler_params=pltpu.CompilerParams(
        dimension_semantics=("arbitrary",),
    ),
)


def kernel(sentences1, sentences2, emb_table):
    sents = jnp.concatenate([sentences1, sentences2], axis=0).reshape(
        NW, IDX_ROWS_PER_W, IDX_COLS
    )
    tail = lax.slice(emb_table, (TAIL_BASE, 0), (VOCAB, EMB))
    tail = jnp.pad(tail, ((0, 0), (0, EMBP - EMB))).reshape(TAIL_N * EMBP)
    table_lin = _sc_detile(emb_table.T, tail)
    pooled = _sc_pool(sents, table_lin.reshape(VOCAB, EMBP))
    row0 = lax.slice(emb_table, (0, 0), (1, EMB))
    loss = _tc_loss(pooled[:B], pooled[B:], sentences1, sentences2, row0)
    return loss[0, 0]


# speedup vs baseline: 2.2790x; 1.0940x over previous
import functools

import jax
import jax.numpy as jnp
from jax import lax
from jax.experimental import pallas as pl
from jax.experimental.pallas import tpu as pltpu
from jax.experimental.pallas import tpu_sc as plsc

VOCAB = 1000000
EMB = 32
EMBP = 40
B = 4096
L = 50
TEMP = 0.05

NC, NS = 2, 16
NW = NC * NS
SENTS = 2 * B
S_PER_W = SENTS // NW
IDX_PER_W = S_PER_W * L
IDX_COLS = 64
IDX_ROWS_PER_W = IDX_PER_W // IDX_COLS
CHUNK_S = 32
CHUNK_I = CHUNK_S * L
GROUPS = CHUNK_I // IDX_COLS
N_CHUNKS = S_PER_W // CHUNK_S

LANES = 16
SLAB = 512
N_SLABS = (VOCAB // 128) // (SLAB // 128)
SLABS_PER_W = N_SLABS // NW
TAIL_BASE = (VOCAB // 128) * 128
TAIL_N = VOCAB - TAIL_BASE


def _make_sc_detile():
    mesh = plsc.VectorSubcoreMesh(
        core_axis_name="c", subcore_axis_name="s", num_cores=NC, num_subcores=NS
    )

    @functools.partial(
        pl.kernel,
        out_type=jax.ShapeDtypeStruct((VOCAB * EMBP,), jnp.float32),
        mesh=mesh,
        scratch_types=[
            pltpu.VMEM((EMB, SLAB), jnp.float32),
            pltpu.VMEM((EMB, SLAB), jnp.float32),
            pltpu.VMEM((SLAB * EMBP,), jnp.float32),
            pltpu.VMEM((SLAB * EMBP,), jnp.float32),
            pltpu.SemaphoreType.DMA,
            pltpu.SemaphoreType.DMA,
            pltpu.SemaphoreType.DMA,
            pltpu.SemaphoreType.DMA,
        ],
        compiler_params=pltpu.CompilerParams(
            use_tc_tiling_on_sc=True, needs_layout_passes=False
        ),
    )
    def sc_detile(tt_hbm, tail_hbm, out_hbm, in_a, in_b, out_a, out_b,
                  si_a, si_b, so_a, so_b):
        wid = lax.axis_index("s") * NC + lax.axis_index("c")
        iota = lax.iota(jnp.int32, LANES) * EMBP

        def slab_of(k):
            return wid + NW * k

        def fire_in(buf, sem, k):
            pltpu.async_copy(
                tt_hbm.at[:, pl.ds(slab_of(k) * SLAB, SLAB)], buf, sem
            )

        def wait_in(buf, sem):
            pltpu.make_async_copy(tt_hbm.at[:, pl.ds(0, SLAB)], buf, sem).wait()

        def transpose(buf, obuf):
            def per_dim(d, _):
                for g0 in range(0, SLAB // LANES, 8):
                    vals = [buf[d, pl.ds((g0 + j) * LANES, LANES)] for j in range(8)]
                    idxs = [iota + ((g0 + j) * LANES * EMBP + d) for j in range(8)]
                    for j in range(8):
                        plsc.store_scatter(obuf, [idxs[j]], vals[j])
                return _
            lax.fori_loop(0, EMB, per_dim, 0)

        def fire_out(obuf, sem, k):
            pltpu.async_copy(
                obuf, out_hbm.at[pl.ds(slab_of(k) * SLAB * EMBP, SLAB * EMBP)], sem
            )

        def wait_out(obuf, sem):
            pltpu.make_async_copy(
                obuf, out_hbm.at[pl.ds(0, SLAB * EMBP)], sem
            ).wait()

        fire_in(in_a, si_a, 0)

        def body(m, _):
            fire_in(in_b, si_b, 2 * m + 1)
            wait_in(in_a, si_a)

            @pl.when(m > 0)
            def _w():
                wait_out(out_a, so_a)

            transpose(in_a, out_a)
            fire_out(out_a, so_a, 2 * m)

            @pl.when(m < (SLABS_PER_W - 1) // 2 - 1)
            def _f():
                fire_in(in_a, si_a, 2 * m + 2)

            wait_in(in_b, si_b)

            @pl.when(m > 0)
            def _w2():
                wait_out(out_b, so_b)

            transpose(in_b, out_b)
            fire_out(out_b, so_b, 2 * m + 1)
            return _

        lax.fori_loop(0, (SLABS_PER_W - 1) // 2, body, 0)

        fire_in(in_a, si_a, SLABS_PER_W - 1)
        wait_in(in_a, si_a)
        wait_out(out_a, so_a)
        transpose(in_a, out_a)
        fire_out(out_a, so_a, SLABS_PER_W - 1)
        wait_out(out_a, so_a)
        wait_out(out_b, so_b)

        @pl.when(wid == 0)
        def _extra():
            pltpu.sync_copy(
                tt_hbm.at[:, pl.ds((N_SLABS - 1) * SLAB, SLAB)], in_a
            )
            transpose(in_a, out_a)
            pltpu.sync_copy(
                out_a, out_hbm.at[pl.ds((N_SLABS - 1) * SLAB * EMBP, SLAB * EMBP)]
            )

        @pl.when(wid == 1)
        def _tail():
            pltpu.sync_copy(tail_hbm, out_b.at[pl.ds(0, TAIL_N * EMBP)])
            pltpu.sync_copy(
                out_b.at[pl.ds(0, TAIL_N * EMBP)],
                out_hbm.at[pl.ds(TAIL_BASE * EMBP, TAIL_N * EMBP)],
            )

    return sc_detile


_sc_detile = _make_sc_detile()


def _make_sc_pool():
    mesh = plsc.VectorSubcoreMesh(
        core_axis_name="c", subcore_axis_name="s", num_cores=NC, num_subcores=NS
    )

    @functools.partial(
        pl.kernel,
        out_type=jax.ShapeDtypeStruct((SENTS, EMB), jnp.float32),
        mesh=mesh,
        scratch_types=[
            pltpu.VMEM((IDX_ROWS_PER_W, IDX_COLS), jnp.int32),
            pltpu.VMEM((CHUNK_I, EMBP), jnp.float32),
            pltpu.VMEM((S_PER_W, EMB), jnp.float32),
            pltpu.SemaphoreType.DMA,
        ],
        compiler_params=pltpu.CompilerParams(use_tc_tiling_on_sc=False),
    )
    def sc_pool(sents_hbm, table_hbm, out_hbm, idx_v, rows_v, acc_v, sem):
        wid = lax.axis_index("s") * NC + lax.axis_index("c")
        pltpu.sync_copy(sents_hbm.at[wid], idx_v)
        for k in range(N_CHUNKS):
            def fire(j, _):
                pltpu.async_copy(
                    table_hbm.at[idx_v.at[k * GROUPS + j]],
                    rows_v.at[pl.ds(j * IDX_COLS, IDX_COLS)],
                    sem,
                )
                return _

            lax.fori_loop(0, GROUPS, fire, 0)
            pltpu.make_async_copy(table_hbm.at[pl.ds(0, CHUNK_I)], rows_v, sem).wait()

            def per_sentence(s, _):
                r0 = s * L
                a0 = rows_v[r0, 0:16]
                a1 = rows_v[r0, 16:32]
                b0 = rows_v[r0 + 1, 0:16]
                b1 = rows_v[r0 + 1, 16:32]
                for j in range(2, L, 2):
                    a0 = a0 + rows_v[r0 + j, 0:16]
                    a1 = a1 + rows_v[r0 + j, 16:32]
                    b0 = b0 + rows_v[r0 + j + 1, 0:16]
                    b1 = b1 + rows_v[r0 + j + 1, 16:32]
                acc_v[k * CHUNK_S + s, 0:16] = a0 + b0
                acc_v[k * CHUNK_S + s, 16:32] = a1 + b1
                return _

            lax.fori_loop(0, CHUNK_S, per_sentence, 0)
        pltpu.sync_copy(acc_v, out_hbm.at[pl.ds(wid * S_PER_W, S_PER_W)])

    return sc_pool


_sc_pool = _make_sc_pool()

BLK = 256


def _tc_loss_body(p1_ref, p2_ref, s1_ref, s2_ref, row0_ref, out_ref):
    i = pl.program_id(0)
    row0 = row0_ref[...]

    cnt2 = jnp.sum((s2_ref[...] == 0).astype(jnp.float32), axis=1, keepdims=True)
    e2 = p2_ref[...] - cnt2 * row0
    n2 = e2 * lax.rsqrt(
        jnp.maximum(jnp.sum(e2 * e2, axis=1, keepdims=True), 1e-24)
    )

    cnt1 = jnp.sum((s1_ref[...] == 0).astype(jnp.float32), axis=1, keepdims=True)
    e1 = p1_ref[...] - cnt1 * row0
    n1 = e1 * lax.rsqrt(
        jnp.maximum(jnp.sum(e1 * e1, axis=1, keepdims=True), 1e-24)
    )

    logits = lax.dot_general(
        n1, n2, (((1,), (1,)), ((), ())),
        precision=lax.Precision.HIGHEST,
        preferred_element_type=jnp.float32,
    ) * (1.0 / TEMP)
    logz = jnp.log(jnp.sum(jnp.exp(logits), axis=1))
    col = lax.broadcasted_iota(jnp.int32, logits.shape, 1)
    row = lax.broadcasted_iota(jnp.int32, logits.shape, 0) + i * BLK
    diag = jnp.sum(jnp.where(col == row, logits, 0.0), axis=1)
    part = jnp.sum(logz - diag).reshape(1, 1) * (1.0 / B)

    @pl.when(i == 0)
    def _():
        out_ref[...] = jnp.zeros((1, 1), jnp.float32)

    out_ref[...] += part


_tc_loss = pl.pallas_call(
    _tc_loss_body,
    grid=(B // BLK,),
    in_specs=[
        pl.BlockSpec((BLK, EMB), lambda i: (i, 0)),
        pl.BlockSpec((B, EMB), lambda i: (0, 0)),
        pl.BlockSpec((BLK, L), lambda i: (i, 0)),
        pl.BlockSpec((B, L), lambda i: (0, 0)),
        pl.BlockSpec((1, EMB), lambda i: (0, 0)),
    ],
    out_specs=pl.BlockSpec((1, 1), lambda i: (0, 0)),
    out_shape=jax.ShapeDtypeStruct((1, 1), jnp.float32),
    compiler_params=pltpu.CompilerParams(
        dimension_semantics=("arbitrary",),
    ),
)


def kernel(sentences1, sentences2, emb_table):
    sents = jnp.concatenate([sentences1, sentences2], axis=0).reshape(
        NW, IDX_ROWS_PER_W, IDX_COLS
    )
    tail = lax.slice(emb_table, (TAIL_BASE, 0), (VOCAB, EMB))
    tail = jnp.pad(tail, ((0, 0), (0, EMBP - EMB))).reshape(TAIL_N * EMBP)
    table_lin = _sc_detile(emb_table.T, tail)
    pooled = _sc_pool(sents, table_lin.reshape(VOCAB, EMBP))
    row0 = lax.slice(emb_table, (0, 0), (1, EMB))
    loss = _tc_loss(pooled[:B], pooled[B:], sentences1, sentences2, row0)
    return loss[0, 0]


# speedup vs baseline: 2.9923x; 1.3130x over previous
import functools

import jax
import jax.numpy as jnp
from jax import lax
from jax.experimental import pallas as pl
from jax.experimental.pallas import tpu as pltpu
from jax.experimental.pallas import tpu_sc as plsc

VOCAB = 1000000
EMB = 32
WPACK = EMB // 2
EMBP = 24
B = 4096
L = 50
TEMP = 0.05

NC, NS = 2, 16
NW = NC * NS
SENTS = 2 * B
S_PER_W = SENTS // NW
IDX_PER_W = S_PER_W * L
IDX_COLS = 64
IDX_ROWS_PER_W = IDX_PER_W // IDX_COLS
CHUNK_S = 32
CHUNK_I = CHUNK_S * L
GROUPS = CHUNK_I // IDX_COLS
N_CHUNKS = S_PER_W // CHUNK_S

LANES = 16
SLAB = 512
N_SLABS = (VOCAB // 128) // (SLAB // 128)
SLABS_PER_W = N_SLABS // NW
TAIL_BASE = (VOCAB // 128) * 128
TAIL_N = VOCAB - TAIL_BASE


def _make_sc_detile():
    mesh = plsc.VectorSubcoreMesh(
        core_axis_name="c", subcore_axis_name="s", num_cores=NC, num_subcores=NS
    )

    @functools.partial(
        pl.kernel,
        out_type=jax.ShapeDtypeStruct((VOCAB * EMBP,), jnp.int32),
        mesh=mesh,
        scratch_types=[
            pltpu.VMEM((EMB, SLAB), jnp.float32),
            pltpu.VMEM((EMB, SLAB), jnp.float32),
            pltpu.VMEM((SLAB * EMBP,), jnp.int32),
            pltpu.VMEM((SLAB * EMBP,), jnp.int32),
            pltpu.SemaphoreType.DMA,
            pltpu.SemaphoreType.DMA,
            pltpu.SemaphoreType.DMA,
            pltpu.SemaphoreType.DMA,
        ],
        compiler_params=pltpu.CompilerParams(
            use_tc_tiling_on_sc=True, needs_layout_passes=False
        ),
    )
    def sc_detile(tt_hbm, tail_hbm, out_hbm, in_a, in_b, out_a, out_b,
                  si_a, si_b, so_a, so_b):
        wid = lax.axis_index("s") * NC + lax.axis_index("c")
        iota = lax.iota(jnp.int32, LANES) * EMBP

        def slab_of(k):
            return wid + NW * k

        def fire_in(buf, sem, k):
            pltpu.async_copy(
                tt_hbm.at[:, pl.ds(slab_of(k) * SLAB, SLAB)], buf, sem
            )

        def wait_in(buf, sem):
            pltpu.make_async_copy(tt_hbm.at[:, pl.ds(0, SLAB)], buf, sem).wait()

        def transpose(buf, obuf, ngroups=SLAB // LANES):
            def per_pair(dp, _):
                for g0 in range(0, ngroups, 8):
                    gs = [g0 + j for j in range(min(8, ngroups - g0))]
                    evens = [buf[2 * dp, pl.ds(g * LANES, LANES)] for g in gs]
                    odds = [buf[2 * dp + 1, pl.ds(g * LANES, LANES)] for g in gs]
                    words = [
                        plsc.bitcast(
                            plsc.pack(e, o, format=plsc.PackFormat.INTERLEAVED),
                            jnp.int32,
                        )
                        for e, o in zip(evens, odds)
                    ]
                    idxs = [iota + (g * LANES * EMBP + dp) for g in gs]
                    for w, ix in zip(words, idxs):
                        plsc.store_scatter(obuf, [ix], w)
                return _
            lax.fori_loop(0, WPACK, per_pair, 0)

        def fire_out(obuf, sem, k):
            pltpu.async_copy(
                obuf, out_hbm.at[pl.ds(slab_of(k) * SLAB * EMBP, SLAB * EMBP)], sem
            )

        def wait_out(obuf, sem):
            pltpu.make_async_copy(
                obuf, out_hbm.at[pl.ds(0, SLAB * EMBP)], sem
            ).wait()

        fire_in(in_a, si_a, 0)

        def body(m, _):
            fire_in(in_b, si_b, 2 * m + 1)
            wait_in(in_a, si_a)

            @pl.when(m > 0)
            def _w():
                wait_out(out_a, so_a)

            transpose(in_a, out_a)
            fire_out(out_a, so_a, 2 * m)

            @pl.when(m < (SLABS_PER_W - 1) // 2 - 1)
            def _f():
                fire_in(in_a, si_a, 2 * m + 2)

            wait_in(in_b, si_b)

            @pl.when(m > 0)
            def _w2():
                wait_out(out_b, so_b)

            transpose(in_b, out_b)
            fire_out(out_b, so_b, 2 * m + 1)
            return _

        lax.fori_loop(0, (SLABS_PER_W - 1) // 2, body, 0)

        fire_in(in_a, si_a, SLABS_PER_W - 1)
        wait_in(in_a, si_a)
        wait_out(out_a, so_a)
        transpose(in_a, out_a)
        fire_out(out_a, so_a, SLABS_PER_W - 1)
        wait_out(out_a, so_a)
        wait_out(out_b, so_b)

        @pl.when(wid == 0)
        def _extra():
            pltpu.sync_copy(
                tt_hbm.at[:, pl.ds((N_SLABS - 1) * SLAB, SLAB)], in_a
            )
            transpose(in_a, out_a)
            pltpu.sync_copy(
                out_a, out_hbm.at[pl.ds((N_SLABS - 1) * SLAB * EMBP, SLAB * EMBP)]
            )

        @pl.when(wid == 1)
        def _tail():
            pltpu.sync_copy(tail_hbm, in_b.at[:, pl.ds(0, 128)])
            transpose(in_b, out_b, ngroups=TAIL_N // LANES)
            pltpu.sync_copy(
                out_b.at[pl.ds(0, TAIL_N * EMBP)],
                out_hbm.at[pl.ds(TAIL_BASE * EMBP, TAIL_N * EMBP)],
            )

    return sc_detile


_sc_detile = _make_sc_detile()


def _make_sc_pool():
    mesh = plsc.VectorSubcoreMesh(
        core_axis_name="c", subcore_axis_name="s", num_cores=NC, num_subcores=NS
    )

    @functools.partial(
        pl.kernel,
        out_type=jax.ShapeDtypeStruct((SENTS, EMB), jnp.float32),
        mesh=mesh,
        scratch_types=[
            pltpu.VMEM((IDX_ROWS_PER_W, IDX_COLS), jnp.int32),
            pltpu.VMEM((CHUNK_I, EMBP), jnp.int32),
            pltpu.VMEM((S_PER_W, EMB), jnp.float32),
            pltpu.SemaphoreType.DMA,
        ],
        compiler_params=pltpu.CompilerParams(
            use_tc_tiling_on_sc=False, needs_layout_passes=False
        ),
    )
    def sc_pool(sents_hbm, table_hbm, out_hbm, idx_v, rows_v, acc_v, sem):
        wid = lax.axis_index("s") * NC + lax.axis_index("c")
        pltpu.sync_copy(sents_hbm.at[wid], idx_v)
        for k in range(N_CHUNKS):
            def fire(j, _):
                pltpu.async_copy(
                    table_hbm.at[idx_v.at[k * GROUPS + j]],
                    rows_v.at[pl.ds(j * IDX_COLS, IDX_COLS)],
                    sem,
                )
                return _

            lax.fori_loop(0, GROUPS, fire, 0)
            pltpu.make_async_copy(table_hbm.at[pl.ds(0, CHUNK_I)], rows_v, sem).wait()

            def unpk(r):
                w = rows_v[r, 0:WPACK]
                return plsc.unpack(
                    plsc.bitcast(w, jnp.bfloat16),
                    format=plsc.PackFormat.INTERLEAVED,
                )

            def per_sentence(s, _):
                r0 = s * L
                a0, a1 = unpk(r0)
                b0, b1 = unpk(r0 + 1)
                for j in range(2, L, 2):
                    e0, e1 = unpk(r0 + j)
                    o0, o1 = unpk(r0 + j + 1)
                    a0 = a0 + e0
                    a1 = a1 + e1
                    b0 = b0 + o0
                    b1 = b1 + o1
                acc_v[k * CHUNK_S + s, 0:16] = a0 + b0
                acc_v[k * CHUNK_S + s, 16:32] = a1 + b1
                return _

            lax.fori_loop(0, CHUNK_S, per_sentence, 0)
        pltpu.sync_copy(acc_v, out_hbm.at[pl.ds(wid * S_PER_W, S_PER_W)])

    return sc_pool


_sc_pool = _make_sc_pool()

BLK = 256


def _tc_loss_body(p1_ref, p2_ref, s1_ref, s2_ref, row0_ref, out_ref):
    i = pl.program_id(0)
    row0 = row0_ref[...]

    cnt2 = jnp.sum((s2_ref[...] == 0).astype(jnp.float32), axis=1, keepdims=True)
    e2 = p2_ref[...] - cnt2 * row0
    n2 = e2 * lax.rsqrt(
        jnp.maximum(jnp.sum(e2 * e2, axis=1, keepdims=True), 1e-24)
    )

    cnt1 = jnp.sum((s1_ref[...] == 0).astype(jnp.float32), axis=1, keepdims=True)
    e1 = p1_ref[...] - cnt1 * row0
    n1 = e1 * lax.rsqrt(
        jnp.maximum(jnp.sum(e1 * e1, axis=1, keepdims=True), 1e-24)
    )

    logits = lax.dot_general(
        n1, n2, (((1,), (1,)), ((), ())),
        precision=lax.Precision.HIGHEST,
        preferred_element_type=jnp.float32,
    ) * (1.0 / TEMP)
    logz = jnp.log(jnp.sum(jnp.exp(logits), axis=1))
    col = lax.broadcasted_iota(jnp.int32, logits.shape, 1)
    row = lax.broadcasted_iota(jnp.int32, logits.shape, 0) + i * BLK
    diag = jnp.sum(jnp.where(col == row, logits, 0.0), axis=1)
    part = jnp.sum(logz - diag).reshape(1, 1) * (1.0 / B)

    @pl.when(i == 0)
    def _():
        out_ref[...] = jnp.zeros((1, 1), jnp.float32)

    out_ref[...] += part


_tc_loss = pl.pallas_call(
    _tc_loss_body,
    grid=(B // BLK,),
    in_specs=[
        pl.BlockSpec((BLK, EMB), lambda i: (i, 0)),
        pl.BlockSpec((B, EMB), lambda i: (0, 0)),
        pl.BlockSpec((BLK, L), lambda i: (i, 0)),
        pl.BlockSpec((B, L), lambda i: (0, 0)),
        pl.BlockSpec((1, EMB), lambda i: (0, 0)),
    ],
    out_specs=pl.BlockSpec((1, 1), lambda i: (0, 0)),
    out_shape=jax.ShapeDtypeStruct((1, 1), jnp.float32),
    compiler_params=pltpu.CompilerParams(
        dimension_semantics=("arbitrary",),
    ),
)


def kernel(sentences1, sentences2, emb_table):
    sents = jnp.concatenate([sentences1, sentences2], axis=0).reshape(
        NW, IDX_ROWS_PER_W, IDX_COLS
    )
    tail_t = lax.slice(emb_table, (TAIL_BASE, 0), (VOCAB, EMB)).T
    tail_t = jnp.pad(tail_t, ((0, 0), (0, 128 - TAIL_N)))
    table_lin = _sc_detile(emb_table.T, tail_t)
    pooled = _sc_pool(sents, table_lin.reshape(VOCAB, EMBP))
    row0 = lax.slice(emb_table, (0, 0), (1, EMB))
    row0 = row0.astype(jnp.bfloat16).astype(jnp.float32)
    row0 = jnp.concatenate([row0[:, 0::2], row0[:, 1::2]], axis=1)
    loss = _tc_loss(pooled[:B], pooled[B:], sentences1, sentences2, row0)
    return loss[0, 0]
